# baseline (speedup 1.0000x reference)
import functools

import jax
import jax.numpy as jnp
from jax import lax
from jax.experimental import pallas as pl
from jax.experimental.pallas import tpu as pltpu
from jax.experimental.pallas import tpu_sc as plsc

N = 10000
F = 128
E = 320000
NC = 2
NS = 16
NW = NC * NS
C = 96
ROWS = 312
ROWS_LAST = 376
DUMP = 10008
AUX0 = 10016
NAUX = 31
NO = 10048
NP = 10064
KMAX = 128
TOT = C * (E // C + 1 + NW + NW * KMAX)
BN_EPS = 1e-5

import numpy as _np

def _slice_bounds():
    esc = E // 2
    w = -(-esc // 240)
    wpt, extra = divmod(w, 16)
    quota = 240 * (wpt + (_np.arange(16) < extra))
    cums = _np.minimum(_np.cumsum(quota), esc)
    return _np.concatenate([cums[:15], [esc], esc + cums[:15]]).astype(_np.int32)

_BOUNDS = _slice_bounds()
_SLICE_OF_POS = _np.searchsorted(_BOUNDS, _np.arange(E), side="right").astype(_np.int32)


def _segment_sum_body(h_hbm, srcp_hbm, dstp_hbm, nch_hbm, starts_hbm,
                      zblk_hbm, out_hbm,
                      sidx, didx, rows, meta, agg_sh, sem):
    c = lax.axis_index("c")
    s = lax.axis_index("s")
    wid = c * NS + s

    pltpu.sync_copy(nch_hbm.at[wid], meta.at[0])
    pltpu.sync_copy(starts_hbm.at[wid], meta.at[1])
    nch = meta[0][0]
    start = meta[1][0]

    roff = pl.multiple_of(wid * ROWS, 8)

    @pl.when(wid < NW - 1)
    def _():
        pltpu.sync_copy(zblk_hbm.at[pl.ds(0, ROWS)],
                        agg_sh.at[pl.ds(roff, ROWS)])

    @pl.when(wid == NW - 1)
    def _():
        pltpu.sync_copy(zblk_hbm, agg_sh.at[pl.ds(roff, ROWS_LAST)])

    plsc.subcore_barrier()

    def body(j, carry):
        off = pl.multiple_of(start + j * C, 8)
        pltpu.sync_copy(srcp_hbm.at[pl.ds(off, C)], sidx)
        pltpu.sync_copy(dstp_hbm.at[pl.ds(off, C)], didx)
        pltpu.async_copy(h_hbm.at[sidx], rows, sem).wait()
        pltpu.sync_copy(rows, agg_sh.at[didx], add=True)
        return carry

    lax.fori_loop(0, nch, body, 0)
    plsc.subcore_barrier()

    @pl.when(wid < NW - 1)
    def _():
        pltpu.sync_copy(agg_sh.at[pl.ds(roff, ROWS)],
                        out_hbm.at[pl.ds(roff, ROWS)])

    @pl.when(wid == NW - 1)
    def _():
        pltpu.sync_copy(agg_sh.at[pl.ds(roff, ROWS_LAST)],
                        out_hbm.at[pl.ds(roff, ROWS_LAST)])


@functools.cache
def _segment_sum_sc():
    mesh = plsc.VectorSubcoreMesh(core_axis_name="c", subcore_axis_name="s",
                                  num_cores=NC, num_subcores=NS)
    return pl.kernel(
        _segment_sum_body,
        out_type=jax.ShapeDtypeStruct((NO, F), jnp.float32),
        mesh=mesh,
        scratch_types=[
            pltpu.VMEM((C,), jnp.int32),
            pltpu.VMEM((C,), jnp.int32),
            pltpu.VMEM((C, F), jnp.float32),
            pltpu.VMEM((2, 16), jnp.int32),
            pltpu.VMEM_SHARED((NP, F), jnp.float32),
            pltpu.SemaphoreType.DMA,
        ],
    )


def _mm1_body(h_ref, agg_ref, w1_ref, b1_ref, o_ref):
    z = h_ref[...] + agg_ref[...]
    a = jnp.dot(z, w1_ref[...], preferred_element_type=jnp.float32)
    o_ref[...] = jnp.maximum(a + b1_ref[...], 0.0)


_mm1 = pl.pallas_call(
    _mm1_body,
    out_shape=jax.ShapeDtypeStruct((N, F), jnp.float32),
)


def _mm2_body(a_ref, m_ref, v_ref, g_ref, b_ref, w2_ref, b2_ref, o_ref):
    bn = (g_ref[...] * (a_ref[...] - m_ref[...])
          / jnp.sqrt(v_ref[...] + BN_EPS) + b_ref[...])
    o = jnp.dot(bn, w2_ref[...], preferred_element_type=jnp.float32)
    o_ref[...] = jnp.maximum(o + b2_ref[...], 0.0)


_mm2 = pl.pallas_call(
    _mm2_body,
    out_shape=jax.ShapeDtypeStruct((N, F), jnp.float32),
)


def _head_body(h_ref, w1_ref, b1_ref, w2_ref, b2_ref, o_ref):
    a = jnp.dot(h_ref[...], w1_ref[...], preferred_element_type=jnp.float32)
    a = jnp.maximum(a + b1_ref[...], 0.0)
    o = jnp.dot(a, w2_ref[...], preferred_element_type=jnp.float32)
    o_ref[...] = o + b2_ref[...]


_head = pl.pallas_call(
    _head_body,
    out_shape=jax.ShapeDtypeStruct((N, F), jnp.float32),
)


def _stable_radix_pass(perm, key, nbuckets):
    onehot = key[:, None] == jnp.arange(nbuckets, dtype=jnp.int32)[None, :]
    csum = jnp.cumsum(onehot.astype(jnp.int32), axis=0)
    rank = jnp.take_along_axis(csum, key[:, None], axis=1)[:, 0] - 1
    base = jnp.concatenate(
        [jnp.zeros((1,), jnp.int32), jnp.cumsum(csum[-1])[:-1].astype(jnp.int32)])
    pos = base[key] + rank
    return jnp.zeros((E,), jnp.int32).at[pos].set(perm)


def _bin_edges(edge_index):
    src, dst = edge_index[0], edge_index[1]
    e_ids = jnp.arange(E, dtype=jnp.int32)
    o1 = _stable_radix_pass(e_ids, dst % 128, 128)
    o2 = _stable_radix_pass(o1, dst[o1] // 128, (N + 127) // 128)
    ds = dst[o2]
    ss = src[o2]

    deg = jnp.bincount(dst, length=N).astype(jnp.int32)
    segstart = jnp.concatenate(
        [jnp.zeros((1,), jnp.int32), jnp.cumsum(deg)[:-1].astype(jnp.int32)])
    mpos = jnp.asarray(_SLICE_OF_POS)
    mfirst = mpos[segstart]
    is_b = mpos > mfirst[ds]
    dsr = jnp.where(is_b, AUX0 + mpos - 1, ds).astype(jnp.int32)

    deg2 = jnp.bincount(dsr, length=NO).astype(jnp.int32)
    segstart2 = jnp.zeros((NO,), jnp.int32).at[:N].set(segstart)
    segstart2 = segstart2.at[AUX0:AUX0 + NAUX].set(jnp.asarray(_BOUNDS))
    k = jnp.minimum(jnp.arange(E, dtype=jnp.int32) - segstart2[dsr], KMAX - 1)

    t = jnp.minimum(dsr // ROWS, NW - 1).astype(jnp.int32)
    rowtile = jnp.minimum(jnp.arange(NO, dtype=jnp.int32) // ROWS, NW - 1)

    act = (deg2[:, None] > jnp.arange(KMAX, dtype=jnp.int32)[None, :])
    ainc = jnp.cumsum(act.astype(jnp.int32), axis=0)
    tstart = jnp.arange(NW, dtype=jnp.int32) * ROWS
    tilebase = jnp.where((tstart > 0)[:, None],
                         ainc[jnp.maximum(tstart - 1, 0)], 0)
    aexc = ainc - act.astype(jnp.int32) - tilebase[rowtile]
    p = aexc[dsr, k]

    ntr = jnp.zeros((NW, KMAX), jnp.int32).at[t, k].add(1)
    cpr = (ntr + (C - 1)) // C
    cb = jnp.cumsum(cpr, axis=1)
    chunkbase = cb - cpr
    tile_nch = cb[:, -1]
    tile_chunkstart = jnp.concatenate(
        [jnp.zeros((1,), jnp.int32),
         jnp.cumsum(tile_nch)[:-1].astype(jnp.int32)])

    chunkidx = chunkbase[t, k] + p // C
    pos = (tile_chunkstart[t] + chunkidx) * C + p % C
    srcp = jnp.zeros((TOT,), jnp.int32).at[pos].set(ss)
    dstp = jnp.full((TOT,), DUMP, jnp.int32).at[pos].set(dsr)
    nch = jnp.tile(tile_nch[:, None], (1, 16))
    starts = jnp.tile((tile_chunkstart * C)[:, None], (1, 16))
    bpos = jnp.asarray(_BOUNDS)
    bvalid = ds[bpos] == ds[bpos - 1]
    bdst = jnp.where(bvalid, ds[bpos], 0)
    return srcp, dstp, nch, starts, bdst, bvalid


def kernel(x, edge_index, params):
    srcp, dstp, nch, starts, bdst, bvalid = _bin_edges(edge_index)
    zblk = jnp.zeros((ROWS_LAST, F), jnp.float32)
    bidx = jnp.where(bvalid, bdst, 0)

    h = x
    for i in range(1, 5):
        p = params[f"conv{i}"]
        out = _segment_sum_sc()(h, srcp, dstp, nch, starts, zblk)
        aux = jnp.where(bvalid[:, None], out[AUX0:AUX0 + NAUX], -0.0)
        agg = out[:N].at[bidx].add(aux)
        a = _mm1(h, agg, p["w1"], p["b1"].reshape(1, F))
        m = jnp.mean(a, axis=0)
        v = jnp.var(a, axis=0)
        h = _mm2(a, m.reshape(1, F), v.reshape(1, F), p["g"].reshape(1, F),
                 p["b"].reshape(1, F), p["w2"], p["b2"].reshape(1, F))

    fc2_w = jnp.pad(params["fc2_w"], ((0, 0), (0, F - 1)))
    fc2_b = jnp.pad(params["fc2_b"].reshape(1, 1), ((0, 0), (0, F - 1)))
    out = _head(h, params["fc1_w"], params["fc1_b"].reshape(1, F),
                fc2_w, fc2_b)
    return out[:, :1]
